# baseline (speedup 1.0000x reference)
import jax
import jax.numpy as jnp
from jax import lax
from jax.experimental import pallas as pl
from jax.experimental.pallas import tpu as pltpu
from jax.experimental.pallas import tpu_sc as plsc

B, F_IN, F_TGT, C, R = 4096, 480, 512, 8, 4
M = C * R
L = 16
TL = 128
BH = B // TL
FH = F_TGT // TL
BT = 64
ST = BT + 1
N_BT = B // BT
BSUB = 32
N_SUB = BT // BSUB
JG = F_TGT // L


def _pack_body(ll_hbm, perm_hbm, out_hbm,
               perm_v, srct_v, tile0, tile1, outb0, outb1,
               semt0, semt1, semo0, semo1):
    m = lax.axis_index("s") * 2 + lax.axis_index("c")
    c = m // R
    r = m % R

    pltpu.sync_copy(perm_hbm, perm_v)

    for t in range(JG):
        v = perm_v[pl.ds(t * L, L)]
        srct_v[pl.ds(t * L, L)] = jnp.where(v < F_IN, v, F_IN)

    zf = jnp.zeros((L,), jnp.float32)
    for buf in (tile0, tile1):
        for q in range(BT // L):
            buf[F_IN, pl.ds(q * L, L)] = zf

    FCH = F_IN // 4

    def stage_copies(u, tile, semt):
        return [pltpu.make_async_copy(
            ll_hbm.at[pl.ds(k * FCH, FCH), c, u // 2, r,
                      pl.ds((u % 2) * BT, BT)],
            tile.at[pl.ds(k * FCH, FCH), pl.ds(0, BT)], semt)
            for k in range(4)]

    def stage_start(u, tile, semt):
        for cp in stage_copies(u, tile, semt):
            cp.start()

    def stage_wait(u, tile, semt):
        for cp in stage_copies(u, tile, semt):
            cp.wait()

    def flush(gp, outb, semo):
        b0 = gp * BSUB
        return pltpu.make_async_copy(
            outb,
            out_hbm.at[pl.ds(b0, BSUB), c, pl.ds(0, FH), r, pl.ds(0, TL)],
            semo)

    def do_tile(u, tile, semt, tile_next, semt_next):
        @pl.when(u < N_BT - 1)
        def _():
            stage_start(u + 1, tile_next, semt_next)

        stage_wait(u, tile, semt)

        for s in range(N_SUB):
            gp = u * N_SUB + s
            outb = (outb0, outb1)[s % 2]
            semo = (semo0, semo1)[s % 2]

            @pl.when(gp >= 2)
            def _():
                flush(gp - 2, outb, semo).wait()

            cb = jnp.full((L,), s * BSUB, jnp.int32)

            @plsc.parallel_loop(0, JG, 1, unroll=3)
            def _(g):
                ridx = srct_v[pl.ds(g * L, L)]
                for bi in range(BSUB):
                    outb[bi, g // 8, pl.ds((g % 8) * L, L)] = (
                        plsc.load_gather(tile, [ridx, cb + bi]))
            flush(gp, outb, semo).start()

    stage_start(0, tile0, semt0)

    def uloop(u2, carry):
        do_tile(2 * u2, tile0, semt0, tile1, semt1)
        do_tile(2 * u2 + 1, tile1, semt1, tile0, semt0)
        return carry

    lax.fori_loop(0, N_BT // 2, uloop, 0)

    flush(N_BT * N_SUB - 2, outb0, semo0).wait()
    flush(N_BT * N_SUB - 1, outb1, semo1).wait()


def kernel(ll, perm):
    llv = (ll.transpose(1, 2, 3, 0)
             .reshape(F_IN, C, R, BH, TL)
             .transpose(0, 1, 3, 2, 4))
    mesh = plsc.VectorSubcoreMesh(core_axis_name="c", subcore_axis_name="s")
    out5 = pl.kernel(
        _pack_body,
        mesh=mesh,
        compiler_params=pltpu.CompilerParams(
            use_tc_tiling_on_sc=False, needs_layout_passes=False),
        out_type=jax.ShapeDtypeStruct((B, C, FH, R, TL), jnp.float32),
        scratch_types=[
            pltpu.VMEM((F_TGT,), jnp.int32),
            pltpu.VMEM((F_TGT,), jnp.int32),
            pltpu.VMEM((F_IN + 1, ST), jnp.float32),
            pltpu.VMEM((F_IN + 1, ST), jnp.float32),
            pltpu.VMEM((BSUB, FH, TL), jnp.float32),
            pltpu.VMEM((BSUB, FH, TL), jnp.float32),
            pltpu.SemaphoreType.DMA,
            pltpu.SemaphoreType.DMA,
            pltpu.SemaphoreType.DMA,
            pltpu.SemaphoreType.DMA,
        ],
    )(llv, perm)
    return (out5.transpose(0, 2, 4, 1, 3)
                .reshape(B, F_TGT, C, R))


# speedup vs baseline: 1.2899x; 1.2899x over previous
import jax
import jax.numpy as jnp
from jax import lax
from jax.experimental import pallas as pl
from jax.experimental.pallas import tpu as pltpu
from jax.experimental.pallas import tpu_sc as plsc

B, F_IN, F_TGT, C, R = 4096, 480, 512, 8, 4
M = C * R
L = 16
TL = 128
BH = B // TL
FH = F_TGT // TL
BT = 64
ST = BT + 1
N_BT = B // BT
BSUB = 32
N_SUB = BT // BSUB
JG = F_TGT // L


def _pack_body(ll_hbm, perm_hbm, out_hbm,
               perm_v, srct_v, tile0, tile1, outb0, outb1,
               semt0, semt1, semo0, semo1):
    m = lax.axis_index("s") * 2 + lax.axis_index("c")
    c = m // R
    r = m % R

    pltpu.sync_copy(perm_hbm, perm_v)

    for t in range(JG):
        v = perm_v[pl.ds(t * L, L)]
        srct_v[pl.ds(t * L, L)] = jnp.where(v < F_IN, v, F_IN)

    zf = jnp.zeros((L,), jnp.float32)
    for buf in (tile0, tile1):
        for q in range(BT // L):
            buf[F_IN, pl.ds(q * L, L)] = zf

    FCH = F_IN // 4

    def stage_copies(u, tile, semt):
        return [pltpu.make_async_copy(
            ll_hbm.at[pl.ds(k * FCH, FCH), c, u // 2, r,
                      pl.ds((u % 2) * BT, BT)],
            tile.at[pl.ds(k * FCH, FCH), pl.ds(0, BT)], semt)
            for k in range(4)]

    def stage_start(u, tile, semt):
        for cp in stage_copies(u, tile, semt):
            cp.start()

    def stage_wait(u, tile, semt):
        for cp in stage_copies(u, tile, semt):
            cp.wait()

    def flush(gp, outb, semo):
        b0 = gp * BSUB
        return pltpu.make_async_copy(
            outb,
            out_hbm.at[pl.ds(b0, BSUB), c, pl.ds(0, FH), r, pl.ds(0, TL)],
            semo)

    def do_tile(u, tile, semt, tile_next, semt_next):
        @pl.when(u < N_BT - 1)
        def _():
            stage_start(u + 1, tile_next, semt_next)

        stage_wait(u, tile, semt)

        for s in range(N_SUB):
            gp = u * N_SUB + s
            outb = (outb0, outb1)[s % 2]
            semo = (semo0, semo1)[s % 2]

            @pl.when(gp >= 2)
            def _():
                flush(gp - 2, outb, semo).wait()

            cb = jnp.full((L,), s * BSUB, jnp.int32)

            @plsc.parallel_loop(0, JG, 1, unroll=2)
            def _(g):
                ridx = srct_v[pl.ds(g * L, L)]
                for bi in range(BSUB):
                    outb[bi, g // 8, pl.ds((g % 8) * L, L)] = (
                        plsc.load_gather(tile, [ridx, cb + bi]))
            flush(gp, outb, semo).start()

    stage_start(0, tile0, semt0)

    def uloop(u2, carry):
        do_tile(2 * u2, tile0, semt0, tile1, semt1)
        do_tile(2 * u2 + 1, tile1, semt1, tile0, semt0)
        return carry

    lax.fori_loop(0, N_BT // 2, uloop, 0)

    flush(N_BT * N_SUB - 2, outb0, semo0).wait()
    flush(N_BT * N_SUB - 1, outb1, semo1).wait()


def kernel(ll, perm):
    llv = (ll.transpose(1, 2, 3, 0)
             .reshape(F_IN, C, R, BH, TL)
             .transpose(0, 1, 3, 2, 4))
    mesh = plsc.VectorSubcoreMesh(core_axis_name="c", subcore_axis_name="s")
    out5 = pl.kernel(
        _pack_body,
        mesh=mesh,
        compiler_params=pltpu.CompilerParams(
            use_tc_tiling_on_sc=False, needs_layout_passes=False),
        out_type=jax.ShapeDtypeStruct((B, C, FH, R, TL), jnp.float32),
        scratch_types=[
            pltpu.VMEM((F_TGT,), jnp.int32),
            pltpu.VMEM((F_TGT,), jnp.int32),
            pltpu.VMEM((F_IN + 1, ST), jnp.float32),
            pltpu.VMEM((F_IN + 1, ST), jnp.float32),
            pltpu.VMEM((BSUB, FH, TL), jnp.float32),
            pltpu.VMEM((BSUB, FH, TL), jnp.float32),
            pltpu.SemaphoreType.DMA,
            pltpu.SemaphoreType.DMA,
            pltpu.SemaphoreType.DMA,
            pltpu.SemaphoreType.DMA,
        ],
    )(llv, perm)
    return (out5.transpose(0, 2, 4, 1, 3)
                .reshape(B, F_TGT, C, R))
